# baseline (speedup 1.0000x reference)
import functools

import jax
import jax.numpy as jnp
from jax import lax
from jax.experimental import pallas as pl
from jax.experimental.pallas import tpu as pltpu
from jax.experimental.pallas import tpu_sc as plsc

_N = 8192
_M = 65536
_F = 21
_FP = 32
_BQ = 256


def _knn_body(bias_ref, posy_ref, keys_ref, idx_ref, w_ref):
    qlat = posy_ref[:, 0:1]
    qlon = posy_ref[:, 1:2]
    klat = keys_ref[0:1, :]
    klon = keys_ref[1:2, :]
    dlat = qlat - klat
    dlon = qlon - klon
    d2 = dlat * dlat + dlon * dlon

    bias = bias_ref[0]
    big = jnp.float32(jnp.inf)
    v1 = jnp.min(d2, axis=1, keepdims=True)
    gt1 = d2 > v1
    v2 = jnp.min(jnp.where(gt1, d2, big), axis=1, keepdims=True)
    gt2 = d2 > v2
    v3 = jnp.min(jnp.where(gt2, d2, big), axis=1, keepdims=True)

    iota = lax.broadcasted_iota(jnp.int32, d2.shape, 1)
    zero = jnp.zeros_like(iota)
    i1 = jnp.sum(jnp.where(gt1, zero, iota), axis=1, keepdims=True)
    i2 = jnp.sum(jnp.where(gt1 & ~gt2, iota, zero), axis=1, keepdims=True)
    i3 = jnp.sum(jnp.where(gt2 & (d2 <= v3), iota, zero), axis=1,
                 keepdims=True)
    nmax = jnp.int32(d2.shape[1] - 1)
    idx_ref[...] = jnp.concatenate(
        [jnp.minimum(i1, nmax), jnp.minimum(i2, nmax),
         jnp.minimum(i3, nmax)], axis=1)

    w1 = 1.0 / jnp.maximum(v1 + bias, 1e-16)
    w2 = 1.0 / jnp.maximum(v2 + bias, 1e-16)
    w3 = 1.0 / jnp.maximum(v3 + bias, 1e-16)
    rden = 1.0 / (w1 + w2 + w3)
    w_ref[...] = jnp.concatenate(
        [w1 * rden, w2 * rden, w3 * rden], axis=1)


def _tc_knn(x, pos_x, pos_y, k):
    m = pos_y.shape[0]
    n = pos_x.shape[0]
    bias = (jnp.asarray(k, jnp.float32) - 3.0).reshape(1)
    keys = pos_x.T
    grid = (m // _BQ,)
    idx, w = pl.pallas_call(
        _knn_body,
        grid=grid,
        in_specs=[
            pl.BlockSpec(memory_space=pltpu.SMEM),
            pl.BlockSpec((_BQ, 2), lambda i: (i, 0)),
            pl.BlockSpec((2, n), lambda i: (0, 0)),
        ],
        out_specs=[
            pl.BlockSpec((_BQ, 3), lambda i: (i, 0)),
            pl.BlockSpec((_BQ, 3), lambda i: (i, 0)),
        ],
        out_shape=[
            jax.ShapeDtypeStruct((m, 3), jnp.int32),
            jax.ShapeDtypeStruct((m, 3), jnp.float32),
        ],
        compiler_params=pltpu.CompilerParams(
            dimension_semantics=("parallel",)),
    )(bias, pos_y, keys)
    return idx, w


_NW = 32
_QW = _M // _NW
_FC = 8
_NFC = _FP // _FC


def _gather_body(xt_hbm, idx_hbm, w_hbm, out_hbm, xbuf, idxbuf, wbuf, ybuf):
    wid = lax.axis_index("s") * 2 + lax.axis_index("c")
    qbase = wid * _QW
    for j in range(3):
        pltpu.sync_copy(idx_hbm.at[pl.ds(j * _M + qbase, _QW)],
                        idxbuf.at[pl.ds(j * _QW, _QW)])
        pltpu.sync_copy(w_hbm.at[pl.ds(j * _M + qbase, _QW)],
                        wbuf.at[pl.ds(j * _QW, _QW)])

    def chunk(fc):
        pltpu.sync_copy(xt_hbm.at[pl.ds(fc * _FC * _N, _FC * _N)], xbuf)

        def group(g, _):
            qo = g * 16
            ii0 = idxbuf[pl.ds(qo, 16)]
            ii1 = idxbuf[pl.ds(_QW + qo, 16)]
            ii2 = idxbuf[pl.ds(2 * _QW + qo, 16)]
            ww0 = wbuf[pl.ds(qo, 16)]
            ww1 = wbuf[pl.ds(_QW + qo, 16)]
            ww2 = wbuf[pl.ds(2 * _QW + qo, 16)]
            for f in range(_FC):
                base = jnp.int32(f * _N)
                g0 = plsc.load_gather(xbuf, [ii0 + base])
                g1 = plsc.load_gather(xbuf, [ii1 + base])
                g2 = plsc.load_gather(xbuf, [ii2 + base])
                acc = ww0 * g0 + ww1 * g1 + ww2 * g2
                ybuf[pl.ds(f * _QW + qo, 16)] = acc
            return ()

        lax.fori_loop(0, _QW // 16, group, (), unroll=False)
        for f in range(_FC):
            pltpu.sync_copy(
                ybuf.at[pl.ds(f * _QW, _QW)],
                out_hbm.at[pl.ds((fc * _FC + f) * _M + qbase, _QW)])

    for fc in range(_NFC):
        chunk(fc)


def _sc_gather(xt_flat, idx_flat, w_flat):
    mesh = plsc.VectorSubcoreMesh(core_axis_name="c", subcore_axis_name="s")
    f = functools.partial(
        pl.kernel,
        mesh=mesh,
        out_type=jax.ShapeDtypeStruct((_FP * _M,), jnp.float32),
        scratch_types=[
            pltpu.VMEM((_FC * _N,), jnp.float32),
            pltpu.VMEM((3 * _QW,), jnp.int32),
            pltpu.VMEM((3 * _QW,), jnp.float32),
            pltpu.VMEM((_FC * _QW,), jnp.float32),
        ],
        compiler_params=pltpu.CompilerParams(needs_layout_passes=False),
    )(_gather_body)
    return f(xt_flat, idx_flat, w_flat)


def kernel(x, pos_x, pos_y, k):
    m = pos_y.shape[0]
    idx, w = _tc_knn(x, pos_x, pos_y, k)
    xt = jnp.pad(x, ((0, 0), (0, _FP - x.shape[1]))).T.reshape(-1)
    idx_flat = idx.T.reshape(-1)
    w_flat = w.T.reshape(-1)
    yt = _sc_gather(xt, idx_flat, w_flat).reshape(_FP, m)
    b, d = 3, x.shape[1] // 3
    return yt[:_F].reshape(b, d, m).transpose(0, 2, 1)


# speedup vs baseline: 4.2251x; 4.2251x over previous
import jax
import jax.numpy as jnp
from jax import lax
from jax.experimental import pallas as pl
from jax.experimental.pallas import tpu as pltpu

_N = 8192
_M = 65536
_F = 21
_BQ = 256
_S = 1536


def _win_body(bias_ref, start_ref, posyt_ref, keys_ref, x_ref,
              out_ref, flag_ref):
    start = start_ref[pl.program_id(0)]
    bias = bias_ref[0]
    big = jnp.float32(jnp.inf)
    qlat = posyt_ref[0:1, :]
    qlon = posyt_ref[1:2, :]
    kl = keys_ref[pl.ds(start, _S), 0:1]
    kn = keys_ref[pl.ds(start, _S), 1:2]
    dlat = kl - qlat
    dlon = kn - qlon
    d2 = dlat * dlat + dlon * dlon

    v1 = jnp.min(d2, axis=0, keepdims=True)
    v2 = jnp.min(jnp.where(d2 > v1, d2, big), axis=0, keepdims=True)
    v3 = jnp.min(jnp.where(d2 > v2, d2, big), axis=0, keepdims=True)
    w1 = 1.0 / jnp.maximum(v1 + bias, 1e-16)
    w2 = 1.0 / jnp.maximum(v2 + bias, 1e-16)
    w3 = 1.0 / jnp.maximum(v3 + bias, 1e-16)
    rden = 1.0 / (w1 + w2 + w3)
    w_mat = jnp.where(d2 <= v3,
                      rden / jnp.maximum(d2 + bias, 1e-16), 0.0)

    out_ref[...] = jax.lax.dot_general(
        w_mat, x_ref[pl.ds(start, _S), :],
        dimension_numbers=(((0,), (0,)), ((), ())),
        preferred_element_type=jnp.float32,
        precision=jax.lax.Precision.DEFAULT)

    lat_first = jnp.min(keys_ref[pl.ds(start, 8), 0:1], axis=0,
                        keepdims=True)
    lat_last = jnp.max(keys_ref[pl.ds(start + (_S - 8), 8), 0:1], axis=0,
                       keepdims=True)
    dl = jnp.maximum(qlat - lat_first, 0.0)
    dr = jnp.maximum(lat_last - qlat, 0.0)
    bl = jnp.where(start == 0, big, dl * dl)
    br = jnp.where(start + _S == keys_ref.shape[0], big, dr * dr)
    ok = v3 < jnp.minimum(bl, br)
    flag_ref[...] = ok.astype(jnp.float32).reshape(1, 1, _BQ)


def _full_body(bias_ref, posy_ref, keys_ref, x_ref, out_ref):
    qlat = posy_ref[:, 0:1]
    qlon = posy_ref[:, 1:2]
    klat = keys_ref[0:1, :]
    klon = keys_ref[1:2, :]
    dlat = qlat - klat
    dlon = qlon - klon
    d2 = dlat * dlat + dlon * dlon

    bias = bias_ref[0]
    big = jnp.float32(jnp.inf)
    v1 = jnp.min(d2, axis=1, keepdims=True)
    v2 = jnp.min(jnp.where(d2 > v1, d2, big), axis=1, keepdims=True)
    v3 = jnp.min(jnp.where(d2 > v2, d2, big), axis=1, keepdims=True)
    w_mat = jnp.where(d2 <= v3,
                      1.0 / jnp.maximum(d2 + bias, 1e-16), 0.0)
    den = (1.0 / jnp.maximum(v1 + bias, 1e-16)
           + 1.0 / jnp.maximum(v2 + bias, 1e-16)
           + 1.0 / jnp.maximum(v3 + bias, 1e-16))
    num = jax.lax.dot_general(
        w_mat, x_ref[...],
        dimension_numbers=(((1,), (0,)), ((), ())),
        preferred_element_type=jnp.float32,
        precision=jax.lax.Precision.DEFAULT)
    out_ref[...] = num * (1.0 / den)


def _full_kernel(bias, x, pos_x, pos_y):
    m, n, f = pos_y.shape[0], pos_x.shape[0], x.shape[1]
    return pl.pallas_call(
        _full_body,
        grid=(m // _BQ,),
        in_specs=[
            pl.BlockSpec(memory_space=pltpu.SMEM),
            pl.BlockSpec((_BQ, 2), lambda i: (i, 0)),
            pl.BlockSpec((2, n), lambda i: (0, 0)),
            pl.BlockSpec((n, f), lambda i: (0, 0)),
        ],
        out_specs=pl.BlockSpec((_BQ, f), lambda i: (i, 0)),
        out_shape=jax.ShapeDtypeStruct((m, f), jnp.float32),
        compiler_params=pltpu.CompilerParams(
            dimension_semantics=("parallel",)),
    )(bias, pos_y, pos_x.T, x)


def kernel(x, pos_x, pos_y, k):
    m = pos_y.shape[0]
    n = pos_x.shape[0]
    f = x.shape[1]
    nblk = m // _BQ
    bias = (jnp.asarray(k, jnp.float32) - 3.0).reshape(1)

    order = jnp.argsort(pos_x[:, 0])
    keys_s = pos_x[order]
    x_s = x[order]
    blk_lat = pos_y[:: _BQ, 0]
    counts = jnp.searchsorted(keys_s[:, 0], blk_lat).astype(jnp.int32)
    starts = jnp.clip(counts - _S // 2, 0, n - _S) & ~jnp.int32(7)

    out, flags = pl.pallas_call(
        _win_body,
        grid=(nblk,),
        in_specs=[
            pl.BlockSpec(memory_space=pltpu.SMEM),
            pl.BlockSpec(memory_space=pltpu.SMEM),
            pl.BlockSpec((2, _BQ), lambda i: (0, i)),
            pl.BlockSpec((n, 2), lambda i: (0, 0)),
            pl.BlockSpec((n, f), lambda i: (0, 0)),
        ],
        out_specs=[
            pl.BlockSpec((_BQ, f), lambda i: (i, 0)),
            pl.BlockSpec((1, 1, _BQ), lambda i: (i, 0, 0)),
        ],
        out_shape=[
            jax.ShapeDtypeStruct((m, f), jnp.float32),
            jax.ShapeDtypeStruct((nblk, 1, _BQ), jnp.float32),
        ],
        compiler_params=pltpu.CompilerParams(
            dimension_semantics=("arbitrary",)),
    )(bias, starts, pos_y.T, keys_s, x_s)

    all_ok = jnp.all(flags > 0.5)
    out = lax.cond(all_ok,
                   lambda: out,
                   lambda: _full_kernel(bias, x, pos_x, pos_y))

    b, d = 3, f // 3
    return out.reshape(m, b, d).transpose(1, 0, 2)


# speedup vs baseline: 6.7722x; 1.6028x over previous
import jax
import jax.numpy as jnp
from jax import lax
from jax.experimental import pallas as pl
from jax.experimental.pallas import tpu as pltpu

_N = 8192
_M = 65536
_F = 21
_BQ = 512
_S = 1024


def _win_body(bias_ref, start_ref, posyt_ref, keys_ref, x_ref,
              out_ref, flag_ref):
    start = start_ref[pl.program_id(0)]
    bias = bias_ref[0]
    big = jnp.float32(jnp.inf)
    qlat = posyt_ref[0:1, :]
    qlon = posyt_ref[1:2, :]
    kl = keys_ref[pl.ds(start, _S), 0:1]
    kn = keys_ref[pl.ds(start, _S), 1:2]
    dlat = kl - qlat
    dlon = kn - qlon
    d2 = dlat * dlat + dlon * dlon

    v1 = jnp.min(d2, axis=0, keepdims=True)
    v2 = jnp.min(jnp.where(d2 > v1, d2, big), axis=0, keepdims=True)
    v3 = jnp.min(jnp.where(d2 > v2, d2, big), axis=0, keepdims=True)
    w1 = 1.0 / jnp.maximum(v1 + bias, 1e-16)
    w2 = 1.0 / jnp.maximum(v2 + bias, 1e-16)
    w3 = 1.0 / jnp.maximum(v3 + bias, 1e-16)
    rden = 1.0 / (w1 + w2 + w3)
    w_mat = jnp.where(d2 <= v3,
                      rden / jnp.maximum(d2 + bias, 1e-16), 0.0)

    out_ref[...] = jax.lax.dot_general(
        w_mat, x_ref[pl.ds(start, _S), :],
        dimension_numbers=(((0,), (0,)), ((), ())),
        preferred_element_type=jnp.float32,
        precision=jax.lax.Precision.DEFAULT)

    lat_first = jnp.min(keys_ref[pl.ds(start, 8), 0:1], axis=0,
                        keepdims=True)
    lat_last = jnp.max(keys_ref[pl.ds(start + (_S - 8), 8), 0:1], axis=0,
                       keepdims=True)
    dl = jnp.maximum(qlat - lat_first, 0.0)
    dr = jnp.maximum(lat_last - qlat, 0.0)
    bl = jnp.where(start == 0, big, dl * dl)
    br = jnp.where(start + _S == keys_ref.shape[0], big, dr * dr)
    ok = v3 < jnp.minimum(bl, br)
    flag_ref[...] = ok.astype(jnp.float32).reshape(1, 1, _BQ)


def _full_body(bias_ref, posy_ref, keys_ref, x_ref, out_ref):
    qlat = posy_ref[:, 0:1]
    qlon = posy_ref[:, 1:2]
    klat = keys_ref[0:1, :]
    klon = keys_ref[1:2, :]
    dlat = qlat - klat
    dlon = qlon - klon
    d2 = dlat * dlat + dlon * dlon

    bias = bias_ref[0]
    big = jnp.float32(jnp.inf)
    v1 = jnp.min(d2, axis=1, keepdims=True)
    v2 = jnp.min(jnp.where(d2 > v1, d2, big), axis=1, keepdims=True)
    v3 = jnp.min(jnp.where(d2 > v2, d2, big), axis=1, keepdims=True)
    w_mat = jnp.where(d2 <= v3,
                      1.0 / jnp.maximum(d2 + bias, 1e-16), 0.0)
    den = (1.0 / jnp.maximum(v1 + bias, 1e-16)
           + 1.0 / jnp.maximum(v2 + bias, 1e-16)
           + 1.0 / jnp.maximum(v3 + bias, 1e-16))
    num = jax.lax.dot_general(
        w_mat, x_ref[...],
        dimension_numbers=(((1,), (0,)), ((), ())),
        preferred_element_type=jnp.float32,
        precision=jax.lax.Precision.DEFAULT)
    out_ref[...] = num * (1.0 / den)


def _full_kernel(bias, x, pos_x, pos_y):
    m, n, f = pos_y.shape[0], pos_x.shape[0], x.shape[1]
    return pl.pallas_call(
        _full_body,
        grid=(m // _BQ,),
        in_specs=[
            pl.BlockSpec(memory_space=pltpu.SMEM),
            pl.BlockSpec((_BQ, 2), lambda i: (i, 0)),
            pl.BlockSpec((2, n), lambda i: (0, 0)),
            pl.BlockSpec((n, f), lambda i: (0, 0)),
        ],
        out_specs=pl.BlockSpec((_BQ, f), lambda i: (i, 0)),
        out_shape=jax.ShapeDtypeStruct((m, f), jnp.float32),
        compiler_params=pltpu.CompilerParams(
            dimension_semantics=("parallel",)),
    )(bias, pos_y, pos_x.T, x)


def kernel(x, pos_x, pos_y, k):
    m = pos_y.shape[0]
    n = pos_x.shape[0]
    f = x.shape[1]
    nblk = m // _BQ
    bias = (jnp.asarray(k, jnp.float32) - 3.0).reshape(1)

    order = jnp.argsort(pos_x[:, 0])
    keys_s = pos_x[order]
    x_s = x[order]
    blk_lat = pos_y[:: _BQ, 0]
    counts = jnp.searchsorted(keys_s[:, 0], blk_lat).astype(jnp.int32)
    starts = jnp.clip(counts - _S // 2, 0, n - _S) & ~jnp.int32(7)

    out, flags = pl.pallas_call(
        _win_body,
        grid=(nblk,),
        in_specs=[
            pl.BlockSpec(memory_space=pltpu.SMEM),
            pl.BlockSpec(memory_space=pltpu.SMEM),
            pl.BlockSpec((2, _BQ), lambda i: (0, i)),
            pl.BlockSpec((n, 2), lambda i: (0, 0)),
            pl.BlockSpec((n, f), lambda i: (0, 0)),
        ],
        out_specs=[
            pl.BlockSpec((_BQ, f), lambda i: (i, 0)),
            pl.BlockSpec((1, 1, _BQ), lambda i: (i, 0, 0)),
        ],
        out_shape=[
            jax.ShapeDtypeStruct((m, f), jnp.float32),
            jax.ShapeDtypeStruct((nblk, 1, _BQ), jnp.float32),
        ],
        compiler_params=pltpu.CompilerParams(
            dimension_semantics=("arbitrary",)),
    )(bias, starts, pos_y.T, keys_s, x_s)

    all_ok = jnp.all(flags > 0.5)
    out = lax.cond(all_ok,
                   lambda: out,
                   lambda: _full_kernel(bias, x, pos_x, pos_y))

    b, d = 3, f // 3
    return out.reshape(m, b, d).transpose(1, 0, 2)
